# baseline (speedup 1.0000x reference)
import jax
import jax.numpy as jnp
import numpy as np
from jax.experimental import pallas as pl
from jax.experimental.pallas import tpu as pltpu

B = 1; S = 2048; DIM = 1024; NH = 12; G = 4; HPG = 3; DK = 64; DV = 64
L = 32; DST = 16; LSEL = 64; NSEL = 16; WIN = 512
NCMP = (S - L) // DST + 1
NCMP_P = 128
NSB = S // LSEL
HID = max(1, DK // 2)
HALF = DK // 2
TQ = 256
NQT = S // TQ
NPROJ = NH * DK + 6 * G * DK
QC = NH * DK
GD = HPG * DK
SCALE = 1.0 / float(np.sqrt(DK))
NEG = float(np.finfo(np.float32).min)


def _block_map():
    m = np.zeros((NCMP_P, NSB), np.float32)
    for j in range(NCMP):
        toks = np.arange(j * DST, j * DST + L)
        blks = toks // LSEL
        for mm in np.unique(blks):
            m[j, mm] = float(np.mean(blks == mm))
    return m


_BLKMAP = jnp.asarray(_block_map())

_DNT = (((1,), (1,)), ((), ()))


def _msoftmax(s, mask):
    sm = jnp.where(mask, s, NEG)
    mx = jnp.max(sm, axis=-1, keepdims=True)
    e = jnp.where(mask, jnp.exp(sm - mx), 0.0)
    return e / jnp.maximum(jnp.sum(e, axis=-1, keepdims=True), 1e-9)


def _proj_kernel(x_ref, w_ref, cos_ref, sin_ref,
                 q_ref, kvsel_ref, kvwin_ref, kvcmp_ref):
    y = jnp.dot(x_ref[...], w_ref[...], preferred_element_type=jnp.float32)
    cos = cos_ref[...]
    sin = sin_ref[...]

    def rope(seg):
        x1 = seg[:, :HALF]
        x2 = seg[:, HALF:]
        return jnp.concatenate([x1 * cos - x2 * sin, x1 * sin + x2 * cos], axis=1)

    for g in range(G):
        qcols = []
        for h in range(HPG):
            c = (g * HPG + h) * DK
            qcols.append(rope(y[:, c:c + DK]))
        q_ref[g] = jnp.concatenate(qcols, axis=1)
        ks = rope(y[:, QC + g * DK:QC + (g + 1) * DK])
        vs = y[:, QC + (G + g) * DK:QC + (G + g + 1) * DK]
        kvsel_ref[g] = jnp.concatenate([ks, vs], axis=1)
        kw = rope(y[:, QC + (2 * G + g) * DK:QC + (2 * G + g + 1) * DK])
        vw = y[:, QC + (3 * G + g) * DK:QC + (3 * G + g + 1) * DK]
        kvwin_ref[g] = jnp.concatenate([kw, vw], axis=1)
        kcr = rope(y[:, QC + (4 * G + g) * DK:QC + (4 * G + g + 1) * DK])
        vcr = y[:, QC + (5 * G + g) * DK:QC + (5 * G + g + 1) * DK]
        kvcmp_ref[g] = jnp.concatenate([kcr, vcr], axis=1)


def _cmp_kernel(kv_ref, kvc_ref):
    kv = kv_ref[0]
    cs = jnp.sum(kv.reshape(S // DST, DST, 2 * DK), axis=1)
    pair = (cs[:NCMP] + cs[1:NCMP + 1]) * (1.0 / L)
    kvc_ref[0] = jnp.concatenate(
        [pair, jnp.zeros((NCMP_P - NCMP, 2 * DK), jnp.float32)], axis=0)


def _attn_kernel(q_ref, kvsel_ref, kvw0_ref, kvw1_ref, kvw2_ref, kvc_ref,
                 bmap_ref, f1w_ref, f1b_ref, f2w_ref, f2b_ref, o_ref,
                 sbuf_ref, oacc_ref):
    i = pl.program_id(1)
    s0 = i * TQ
    srow = s0 + jax.lax.broadcasted_iota(jnp.int32, (TQ, 1), 0)
    q = q_ref[0]
    kvc = kvc_ref[0]
    kc = kvc[:, :DK]
    vc = kvc[:, DK:]

    jidx = jax.lax.broadcasted_iota(jnp.int32, (TQ, NCMP_P), 1)
    mc = (jidx < NCMP) & (srow >= jidx * DST + L - 1)
    pcs = []
    ocmps = []
    for h in range(HPG):
        qh = q[:, h * DK:(h + 1) * DK]
        sc = jax.lax.dot_general(qh, kc, _DNT,
                                 preferred_element_type=jnp.float32) * SCALE
        pc = _msoftmax(sc, mc)
        pcs.append(pc)
        ocmps.append(jnp.dot(pc, vc, preferred_element_type=jnp.float32))
    p_grp = pcs[0] + pcs[1] + pcs[2]
    p_slc = jnp.dot(p_grp, bmap_ref[...], preferred_element_type=jnp.float32)

    blk = srow // LSEL
    midx = jax.lax.broadcasted_iota(jnp.int32, (TQ, NSB), 1)
    force = (midx == 0) | (midx == blk)
    allowed = midx <= blk
    p_adj = jnp.where(force, p_slc + 1e6, p_slc)
    p_adj = jnp.where(allowed, p_adj, -1e9)
    rank = jnp.zeros((TQ, NSB), jnp.float32)
    for mp in range(NSB):
        v = p_adj[:, mp:mp + 1]
        rank += jnp.where(v > p_adj, 1.0, 0.0)
        rank += jnp.where((v == p_adj) & (midx > mp), 1.0, 0.0)
    selb = (rank < NSEL) & allowed
    selb_f = jnp.where(selb, 1.0, 0.0)

    erow = jax.lax.broadcasted_iota(jnp.int32, (NSB, S), 0)
    ecol = jax.lax.broadcasted_iota(jnp.int32, (NSB, S), 1) // LSEL
    e_f = jnp.where(erow == ecol, 1.0, 0.0)
    tok_f = jnp.dot(selb_f, e_f, preferred_element_type=jnp.float32)

    kcol = jax.lax.broadcasted_iota(jnp.int32, (TQ, S), 1)
    causal = kcol <= srow
    mask_sel = (tok_f > 0.5) & causal

    bidx = [jnp.maximum(i - 2, 0), jnp.maximum(i - 1, 0), i]
    part_valid = [i >= 2, i >= 1, i >= 0]
    wcol0 = jax.lax.broadcasted_iota(jnp.int32, (TQ, TQ), 1)
    wmasks = []
    for p in range(3):
        gcol = bidx[p] * TQ + wcol0
        m = (gcol <= srow) & ((srow - gcol) < WIN)
        wmasks.append(jnp.logical_and(part_valid[p], m))
    mask_win = jnp.concatenate(wmasks, axis=1)

    qp = (q[:, 0:DK] + q[:, DK:2 * DK] + q[:, 2 * DK:3 * DK]) * (1.0 / 3.0)
    h1 = jnp.dot(qp, f1w_ref[...], preferred_element_type=jnp.float32) + f1b_ref[...]
    h1 = h1 * jax.nn.sigmoid(h1)
    gl = jnp.dot(h1, f2w_ref[...], preferred_element_type=jnp.float32) + f2b_ref[...]
    a = gl[:, 0:1]; b = gl[:, 1:2]; c = gl[:, 2:3]
    m1 = jnp.maximum(a, jnp.maximum(b, c))
    am0 = (a >= b) & (a >= c)
    am1 = jnp.logical_not(am0) & (b >= c)
    am2 = jnp.logical_not(am0) & jnp.logical_not(am1)
    m2 = jnp.where(am0, jnp.maximum(b, c),
                   jnp.where(am1, jnp.maximum(a, c), jnp.maximum(a, b)))
    peaked = (m1 - m2) > 50.0
    ea = jnp.exp(a - m1); eb = jnp.exp(b - m1); ec = jnp.exp(c - m1)
    den = ea + eb + ec
    p0 = jnp.where(peaked, jnp.where(am0, 1.0, 0.0), ea / den)
    p1 = jnp.where(peaked, jnp.where(am1, 1.0, 0.0), eb / den)
    p2 = jnp.where(peaked, jnp.where(am2, 1.0, 0.0), ec / den)

    kvw = [kvw0_ref[0], kvw1_ref[0], kvw2_ref[0]]
    kwin = jnp.concatenate([p[:, :DK] for p in kvw], axis=0)
    vwin = jnp.concatenate([p[:, DK:] for p in kvw], axis=0)
    for h in range(HPG):
        qh = q[:, h * DK:(h + 1) * DK]
        for kt in range(NQT):
            @pl.when(kt <= i)
            def _(kt=kt):
                ks = kvsel_ref[0, kt * TQ:(kt + 1) * TQ, :]
                sbuf_ref[:, kt * TQ:(kt + 1) * TQ] = jax.lax.dot_general(
                    qh, ks[:, :DK], _DNT,
                    preferred_element_type=jnp.float32) * SCALE
        psel = _msoftmax(sbuf_ref[...], mask_sel)
        oacc_ref[...] = jnp.zeros((TQ, DV), jnp.float32)
        for kt in range(NQT):
            @pl.when(kt <= i)
            def _(kt=kt):
                vs = kvsel_ref[0, kt * TQ:(kt + 1) * TQ, DK:]
                oacc_ref[...] += jnp.dot(psel[:, kt * TQ:(kt + 1) * TQ], vs,
                                         preferred_element_type=jnp.float32)
        o_sel = oacc_ref[...]
        sw = jax.lax.dot_general(qh, kwin, _DNT,
                                 preferred_element_type=jnp.float32) * SCALE
        pwin = _msoftmax(sw, mask_win)
        o_win = jnp.dot(pwin, vwin, preferred_element_type=jnp.float32)
        o_ref[0, :, h * DV:(h + 1) * DV] = p0 * ocmps[h] + p1 * o_sel + p2 * o_win


def _out_kernel(o0_ref, o1_ref, o2_ref, o3_ref,
                w0_ref, w1_ref, w2_ref, w3_ref, y_ref):
    acc = jnp.dot(o0_ref[0], w0_ref[...], preferred_element_type=jnp.float32)
    acc += jnp.dot(o1_ref[0], w1_ref[...], preferred_element_type=jnp.float32)
    acc += jnp.dot(o2_ref[0], w2_ref[...], preferred_element_type=jnp.float32)
    acc += jnp.dot(o3_ref[0], w3_ref[...], preferred_element_type=jnp.float32)
    y_ref[...] = acc


def kernel(x, WQ, WKsel, WVsel, WKwin, WVwin, WKcmp, WVcmp, Wout, fc1W, fc1b, fc2W, fc2b):
    f32 = jnp.float32
    x2 = x[0]
    wall = jnp.concatenate([WQ, WKsel, WVsel, WKwin, WVwin, WKcmp, WVcmp], axis=0).T

    pos = jnp.arange(S, dtype=f32)
    freqs = 1.0 / (10000.0 ** (jnp.arange(HALF, dtype=f32) / HALF))
    ang = pos[:, None] * freqs[None, :]
    cos = jnp.cos(ang)
    sin = jnp.sin(ang)

    q, kvsel, kvwin, kvcmp = pl.pallas_call(
        _proj_kernel,
        grid=(NQT,),
        in_specs=[
            pl.BlockSpec((TQ, DIM), lambda i: (i, 0)),
            pl.BlockSpec((DIM, NPROJ), lambda i: (0, 0)),
            pl.BlockSpec((TQ, HALF), lambda i: (i, 0)),
            pl.BlockSpec((TQ, HALF), lambda i: (i, 0)),
        ],
        out_specs=[
            pl.BlockSpec((G, TQ, GD), lambda i: (0, i, 0)),
            pl.BlockSpec((G, TQ, 2 * DK), lambda i: (0, i, 0)),
            pl.BlockSpec((G, TQ, 2 * DK), lambda i: (0, i, 0)),
            pl.BlockSpec((G, TQ, 2 * DK), lambda i: (0, i, 0)),
        ],
        out_shape=[
            jax.ShapeDtypeStruct((G, S, GD), f32),
            jax.ShapeDtypeStruct((G, S, 2 * DK), f32),
            jax.ShapeDtypeStruct((G, S, 2 * DK), f32),
            jax.ShapeDtypeStruct((G, S, 2 * DK), f32),
        ],
    )(x2, wall, cos, sin)

    kvc = pl.pallas_call(
        _cmp_kernel,
        grid=(G,),
        in_specs=[
            pl.BlockSpec((1, S, 2 * DK), lambda g: (g, 0, 0)),
        ],
        out_specs=pl.BlockSpec((1, NCMP_P, 2 * DK), lambda g: (g, 0, 0)),
        out_shape=jax.ShapeDtypeStruct((G, NCMP_P, 2 * DK), f32),
    )(kvcmp)

    f1wt = fc1W.T
    f1b2 = fc1b.reshape(1, HID)
    f2wt = jnp.zeros((HID, 8), f32).at[:, :3].set(fc2W.T)
    f2b2 = jnp.zeros((1, 8), f32).at[:, :3].set(fc2b)

    o = pl.pallas_call(
        _attn_kernel,
        grid=(G, NQT),
        in_specs=[
            pl.BlockSpec((1, TQ, GD), lambda g, i: (g, i, 0)),
            pl.BlockSpec((1, S, 2 * DK), lambda g, i: (g, 0, 0)),
            pl.BlockSpec((1, TQ, 2 * DK),
                         lambda g, i: (g, jnp.maximum(i - 2, 0), 0)),
            pl.BlockSpec((1, TQ, 2 * DK),
                         lambda g, i: (g, jnp.maximum(i - 1, 0), 0)),
            pl.BlockSpec((1, TQ, 2 * DK), lambda g, i: (g, i, 0)),
            pl.BlockSpec((1, NCMP_P, 2 * DK), lambda g, i: (g, 0, 0)),
            pl.BlockSpec((NCMP_P, NSB), lambda g, i: (0, 0)),
            pl.BlockSpec((DK, HID), lambda g, i: (0, 0)),
            pl.BlockSpec((1, HID), lambda g, i: (0, 0)),
            pl.BlockSpec((HID, 8), lambda g, i: (0, 0)),
            pl.BlockSpec((1, 8), lambda g, i: (0, 0)),
        ],
        out_specs=pl.BlockSpec((1, TQ, GD), lambda g, i: (g, i, 0)),
        out_shape=jax.ShapeDtypeStruct((G, S, GD), f32),
        scratch_shapes=[
            pltpu.VMEM((TQ, S), f32),
            pltpu.VMEM((TQ, DV), f32),
        ],
    )(q, kvsel, kvwin, kvwin, kvwin, kvc, _BLKMAP, f1wt, f1b2, f2wt, f2b2)

    wout_t = Wout.T
    out = pl.pallas_call(
        _out_kernel,
        grid=(NQT,),
        in_specs=[
            pl.BlockSpec((1, TQ, GD), lambda i: (0, i, 0)),
            pl.BlockSpec((1, TQ, GD), lambda i: (1, i, 0)),
            pl.BlockSpec((1, TQ, GD), lambda i: (2, i, 0)),
            pl.BlockSpec((1, TQ, GD), lambda i: (3, i, 0)),
            pl.BlockSpec((GD, DIM), lambda i: (0, 0)),
            pl.BlockSpec((GD, DIM), lambda i: (1, 0)),
            pl.BlockSpec((GD, DIM), lambda i: (2, 0)),
            pl.BlockSpec((GD, DIM), lambda i: (3, 0)),
        ],
        out_specs=pl.BlockSpec((TQ, DIM), lambda i: (i, 0)),
        out_shape=jax.ShapeDtypeStruct((S, DIM), f32),
    )(o, o, o, o, wout_t, wout_t, wout_t, wout_t)

    return out.reshape(B, S, DIM)


# speedup vs baseline: 1.4950x; 1.4950x over previous
import jax
import jax.numpy as jnp
import numpy as np
from jax.experimental import pallas as pl
from jax.experimental.pallas import tpu as pltpu

B = 1; S = 2048; DIM = 1024; NH = 12; G = 4; HPG = 3; DK = 64; DV = 64
L = 32; DST = 16; LSEL = 64; NSEL = 16; WIN = 512
NCMP = (S - L) // DST + 1
NCMP_P = 128
NSB = S // LSEL
HID = max(1, DK // 2)
HALF = DK // 2
TQ = 256
NQT = S // TQ
NPROJ = NH * DK + 6 * G * DK
QC = NH * DK
GD = HPG * DK
SCALE = 1.0 / float(np.sqrt(DK))
NEG = float(np.finfo(np.float32).min)


def _block_map():
    m = np.zeros((NCMP_P, NSB), np.float32)
    for j in range(NCMP):
        toks = np.arange(j * DST, j * DST + L)
        blks = toks // LSEL
        for mm in np.unique(blks):
            m[j, mm] = float(np.mean(blks == mm))
    return m


_BLKMAP = jnp.asarray(_block_map())

_DNT = (((1,), (1,)), ((), ()))


def _msoftmax(s, mask):
    sm = jnp.where(mask, s, NEG)
    mx = jnp.max(sm, axis=-1, keepdims=True)
    e = jnp.where(mask, jnp.exp(sm - mx), 0.0)
    return e / jnp.maximum(jnp.sum(e, axis=-1, keepdims=True), 1e-9)


def _proj_kernel(x_ref, w_ref, cos_ref, sin_ref,
                 q_ref, kvsel_ref, kvwin_ref, kvcmp_ref):
    y = jnp.dot(x_ref[...], w_ref[...], preferred_element_type=jnp.float32)
    cos = cos_ref[...]
    sin = sin_ref[...]

    def rope(seg):
        x1 = seg[:, :HALF]
        x2 = seg[:, HALF:]
        return jnp.concatenate([x1 * cos - x2 * sin, x1 * sin + x2 * cos], axis=1)

    for g in range(G):
        qcols = []
        for h in range(HPG):
            c = (g * HPG + h) * DK
            qcols.append(rope(y[:, c:c + DK]))
        q_ref[g] = jnp.concatenate(qcols, axis=1)
        ks = rope(y[:, QC + g * DK:QC + (g + 1) * DK])
        vs = y[:, QC + (G + g) * DK:QC + (G + g + 1) * DK]
        kvsel_ref[g] = jnp.concatenate([ks, vs], axis=1)
        kw = rope(y[:, QC + (2 * G + g) * DK:QC + (2 * G + g + 1) * DK])
        vw = y[:, QC + (3 * G + g) * DK:QC + (3 * G + g + 1) * DK]
        kvwin_ref[g] = jnp.concatenate([kw, vw], axis=1)
        kcr = rope(y[:, QC + (4 * G + g) * DK:QC + (4 * G + g + 1) * DK])
        vcr = y[:, QC + (5 * G + g) * DK:QC + (5 * G + g + 1) * DK]
        kvcmp_ref[g] = jnp.concatenate([kcr, vcr], axis=1)


def _cmp_kernel(kv_ref, kvc_ref):
    kv = kv_ref[0]
    cs = jnp.sum(kv.reshape(S // DST, DST, 2 * DK), axis=1)
    pair = (cs[:NCMP] + cs[1:NCMP + 1]) * (1.0 / L)
    kvc_ref[0] = jnp.concatenate(
        [pair, jnp.zeros((NCMP_P - NCMP, 2 * DK), jnp.float32)], axis=0)


def _attn_kernel(q_ref, kvsel_ref, kvw0_ref, kvw1_ref, kvw2_ref, kvc_ref,
                 bmap_ref, f1w_ref, f1b_ref, f2w_ref, f2b_ref, o_ref):
    i = pl.program_id(1)
    s0 = i * TQ
    srow = s0 + jax.lax.broadcasted_iota(jnp.int32, (TQ, 1), 0)
    q = q_ref[0]
    kvc = kvc_ref[0]
    kc = kvc[:, :DK]
    vc = kvc[:, DK:]

    jidx = jax.lax.broadcasted_iota(jnp.int32, (TQ, NCMP_P), 1)
    mc = (jidx < NCMP) & (srow >= jidx * DST + L - 1)
    pcs = []
    ocmps = []
    for h in range(HPG):
        qh = q[:, h * DK:(h + 1) * DK]
        sc = jax.lax.dot_general(qh, kc, _DNT,
                                 preferred_element_type=jnp.float32) * SCALE
        pc = _msoftmax(sc, mc)
        pcs.append(pc)
        ocmps.append(jnp.dot(pc, vc, preferred_element_type=jnp.float32))
    p_grp = pcs[0] + pcs[1] + pcs[2]
    p_slc = jnp.dot(p_grp, bmap_ref[...], preferred_element_type=jnp.float32)

    blk = srow // LSEL
    midx = jax.lax.broadcasted_iota(jnp.int32, (TQ, NSB), 1)
    force = (midx == 0) | (midx == blk)
    allowed = midx <= blk
    p_adj = jnp.where(force, p_slc + 1e6, p_slc)
    p_adj = jnp.where(allowed, p_adj, -1e9)
    rank = jnp.zeros((TQ, NSB), jnp.float32)
    for mp in range(NSB):
        v = p_adj[:, mp:mp + 1]
        rank += jnp.where(v > p_adj, 1.0, 0.0)
        rank += jnp.where((v == p_adj) & (midx > mp), 1.0, 0.0)
    selb = (rank < NSEL) & allowed
    selb_f = jnp.where(selb, 1.0, 0.0)

    erow = jax.lax.broadcasted_iota(jnp.int32, (NSB, S), 0)
    ecol = jax.lax.broadcasted_iota(jnp.int32, (NSB, S), 1) // LSEL
    e_f = jnp.where(erow == ecol, 1.0, 0.0)
    tok_f = jnp.dot(selb_f, e_f, preferred_element_type=jnp.float32)

    kcol = jax.lax.broadcasted_iota(jnp.int32, (TQ, S), 1)
    causal = kcol <= srow
    mask_sel = (tok_f > 0.5) & causal

    bidx = [jnp.maximum(i - 2, 0), jnp.maximum(i - 1, 0), i]
    part_valid = [i >= 2, i >= 1, i >= 0]
    wcol0 = jax.lax.broadcasted_iota(jnp.int32, (TQ, TQ), 1)
    wmasks = []
    for p in range(3):
        gcol = bidx[p] * TQ + wcol0
        m = (gcol <= srow) & ((srow - gcol) < WIN)
        wmasks.append(jnp.logical_and(part_valid[p], m))
    mask_win = jnp.concatenate(wmasks, axis=1)

    qp = (q[:, 0:DK] + q[:, DK:2 * DK] + q[:, 2 * DK:3 * DK]) * (1.0 / 3.0)
    h1 = jnp.dot(qp, f1w_ref[...], preferred_element_type=jnp.float32) + f1b_ref[...]
    h1 = h1 * jax.nn.sigmoid(h1)
    gl = jnp.dot(h1, f2w_ref[...], preferred_element_type=jnp.float32) + f2b_ref[...]
    a = gl[:, 0:1]; b = gl[:, 1:2]; c = gl[:, 2:3]
    m1 = jnp.maximum(a, jnp.maximum(b, c))
    am0 = (a >= b) & (a >= c)
    am1 = jnp.logical_not(am0) & (b >= c)
    am2 = jnp.logical_not(am0) & jnp.logical_not(am1)
    m2 = jnp.where(am0, jnp.maximum(b, c),
                   jnp.where(am1, jnp.maximum(a, c), jnp.maximum(a, b)))
    peaked = (m1 - m2) > 50.0
    ea = jnp.exp(a - m1); eb = jnp.exp(b - m1); ec = jnp.exp(c - m1)
    den = ea + eb + ec
    p0 = jnp.where(peaked, jnp.where(am0, 1.0, 0.0), ea / den)
    p1 = jnp.where(peaked, jnp.where(am1, 1.0, 0.0), eb / den)
    p2 = jnp.where(peaked, jnp.where(am2, 1.0, 0.0), ec / den)

    kvw = [kvw0_ref[0], kvw1_ref[0], kvw2_ref[0]]
    kwin = jnp.concatenate([p[:, :DK] for p in kvw], axis=0)
    vwin = jnp.concatenate([p[:, DK:] for p in kvw], axis=0)
    for h in range(HPG):
        qh = q[:, h * DK:(h + 1) * DK]
        kvsel = kvsel_ref[0]
        ss = jax.lax.dot_general(qh, kvsel[:, :DK], _DNT,
                                 preferred_element_type=jnp.float32) * SCALE
        psel = _msoftmax(ss, mask_sel)
        o_sel = jnp.dot(psel, kvsel[:, DK:], preferred_element_type=jnp.float32)
        sw = jax.lax.dot_general(qh, kwin, _DNT,
                                 preferred_element_type=jnp.float32) * SCALE
        pwin = _msoftmax(sw, mask_win)
        o_win = jnp.dot(pwin, vwin, preferred_element_type=jnp.float32)
        o_ref[0, :, h * DV:(h + 1) * DV] = p0 * ocmps[h] + p1 * o_sel + p2 * o_win


def _out_kernel(o0_ref, o1_ref, o2_ref, o3_ref,
                w0_ref, w1_ref, w2_ref, w3_ref, y_ref):
    acc = jnp.dot(o0_ref[0], w0_ref[...], preferred_element_type=jnp.float32)
    acc += jnp.dot(o1_ref[0], w1_ref[...], preferred_element_type=jnp.float32)
    acc += jnp.dot(o2_ref[0], w2_ref[...], preferred_element_type=jnp.float32)
    acc += jnp.dot(o3_ref[0], w3_ref[...], preferred_element_type=jnp.float32)
    y_ref[...] = acc


def kernel(x, WQ, WKsel, WVsel, WKwin, WVwin, WKcmp, WVcmp, Wout, fc1W, fc1b, fc2W, fc2b):
    f32 = jnp.float32
    x2 = x[0]
    wall = jnp.concatenate([WQ, WKsel, WVsel, WKwin, WVwin, WKcmp, WVcmp], axis=0).T

    pos = jnp.arange(S, dtype=f32)
    freqs = 1.0 / (10000.0 ** (jnp.arange(HALF, dtype=f32) / HALF))
    ang = pos[:, None] * freqs[None, :]
    cos = jnp.cos(ang)
    sin = jnp.sin(ang)

    q, kvsel, kvwin, kvcmp = pl.pallas_call(
        _proj_kernel,
        grid=(NQT,),
        in_specs=[
            pl.BlockSpec((TQ, DIM), lambda i: (i, 0)),
            pl.BlockSpec((DIM, NPROJ), lambda i: (0, 0)),
            pl.BlockSpec((TQ, HALF), lambda i: (i, 0)),
            pl.BlockSpec((TQ, HALF), lambda i: (i, 0)),
        ],
        out_specs=[
            pl.BlockSpec((G, TQ, GD), lambda i: (0, i, 0)),
            pl.BlockSpec((G, TQ, 2 * DK), lambda i: (0, i, 0)),
            pl.BlockSpec((G, TQ, 2 * DK), lambda i: (0, i, 0)),
            pl.BlockSpec((G, TQ, 2 * DK), lambda i: (0, i, 0)),
        ],
        out_shape=[
            jax.ShapeDtypeStruct((G, S, GD), f32),
            jax.ShapeDtypeStruct((G, S, 2 * DK), f32),
            jax.ShapeDtypeStruct((G, S, 2 * DK), f32),
            jax.ShapeDtypeStruct((G, S, 2 * DK), f32),
        ],
    )(x2, wall, cos, sin)

    kvc = pl.pallas_call(
        _cmp_kernel,
        grid=(G,),
        in_specs=[
            pl.BlockSpec((1, S, 2 * DK), lambda g: (g, 0, 0)),
        ],
        out_specs=pl.BlockSpec((1, NCMP_P, 2 * DK), lambda g: (g, 0, 0)),
        out_shape=jax.ShapeDtypeStruct((G, NCMP_P, 2 * DK), f32),
    )(kvcmp)

    f1wt = fc1W.T
    f1b2 = fc1b.reshape(1, HID)
    f2wt = jnp.zeros((HID, 8), f32).at[:, :3].set(fc2W.T)
    f2b2 = jnp.zeros((1, 8), f32).at[:, :3].set(fc2b)

    o = pl.pallas_call(
        _attn_kernel,
        grid=(G, NQT),
        in_specs=[
            pl.BlockSpec((1, TQ, GD), lambda g, i: (g, i, 0)),
            pl.BlockSpec((1, S, 2 * DK), lambda g, i: (g, 0, 0)),
            pl.BlockSpec((1, TQ, 2 * DK),
                         lambda g, i: (g, jnp.maximum(i - 2, 0), 0)),
            pl.BlockSpec((1, TQ, 2 * DK),
                         lambda g, i: (g, jnp.maximum(i - 1, 0), 0)),
            pl.BlockSpec((1, TQ, 2 * DK), lambda g, i: (g, i, 0)),
            pl.BlockSpec((1, NCMP_P, 2 * DK), lambda g, i: (g, 0, 0)),
            pl.BlockSpec((NCMP_P, NSB), lambda g, i: (0, 0)),
            pl.BlockSpec((DK, HID), lambda g, i: (0, 0)),
            pl.BlockSpec((1, HID), lambda g, i: (0, 0)),
            pl.BlockSpec((HID, 8), lambda g, i: (0, 0)),
            pl.BlockSpec((1, 8), lambda g, i: (0, 0)),
        ],
        out_specs=pl.BlockSpec((1, TQ, GD), lambda g, i: (g, i, 0)),
        out_shape=jax.ShapeDtypeStruct((G, S, GD), f32),
    )(q, kvsel, kvwin, kvwin, kvwin, kvc, _BLKMAP, f1wt, f1b2, f2wt, f2b2)

    wout_t = Wout.T
    out = pl.pallas_call(
        _out_kernel,
        grid=(NQT,),
        in_specs=[
            pl.BlockSpec((1, TQ, GD), lambda i: (0, i, 0)),
            pl.BlockSpec((1, TQ, GD), lambda i: (1, i, 0)),
            pl.BlockSpec((1, TQ, GD), lambda i: (2, i, 0)),
            pl.BlockSpec((1, TQ, GD), lambda i: (3, i, 0)),
            pl.BlockSpec((GD, DIM), lambda i: (0, 0)),
            pl.BlockSpec((GD, DIM), lambda i: (1, 0)),
            pl.BlockSpec((GD, DIM), lambda i: (2, 0)),
            pl.BlockSpec((GD, DIM), lambda i: (3, 0)),
        ],
        out_specs=pl.BlockSpec((TQ, DIM), lambda i: (i, 0)),
        out_shape=jax.ShapeDtypeStruct((S, DIM), f32),
    )(o, o, o, o, wout_t, wout_t, wout_t, wout_t)

    return out.reshape(B, S, DIM)
